# baseline (speedup 1.0000x reference)
import functools

import jax
import jax.numpy as jnp
from jax import lax
from jax.experimental import pallas as pl
from jax.experimental.pallas import tpu as pltpu
from jax.experimental.pallas import tpu_sc as plsc

EMBED = 64
NC = 2
NS = 16
NW = NC * NS

CHUNK = 128
NBUF = 5


@functools.lru_cache(maxsize=None)
def _build(n_tokens):
    b_per_w = n_tokens // NW
    n_chunks = b_per_w // CHUNK
    n_super = n_chunks // NBUF
    assert b_per_w * NW == n_tokens
    assert n_chunks * CHUNK == b_per_w
    assert n_super * NBUF == n_chunks
    mesh = plsc.VectorSubcoreMesh(core_axis_name="c", subcore_axis_name="s")

    @functools.partial(
        pl.kernel,
        mesh=mesh,
        out_type=jax.ShapeDtypeStruct((n_tokens, EMBED), jnp.float32),
        scratch_types=[
            pltpu.VMEM((b_per_w,), jnp.int32),
            pltpu.VMEM((NBUF, CHUNK, EMBED), jnp.float32),
        ] + [pltpu.SemaphoreType.DMA] * (2 * NBUF),
        compiler_params=pltpu.CompilerParams(use_tc_tiling_on_sc=False),
    )
    def emb(tok_hbm, table_hbm, out_hbm, idx_v, rows_v, *sems):
        gsem = sems[:NBUF]
        wsem = sems[NBUF:]
        wid = lax.axis_index("s") * NC + lax.axis_index("c")
        base = wid * b_per_w
        pltpu.sync_copy(tok_hbm.at[pl.ds(base, b_per_w)], idx_v)

        def _gather_args(j, b):
            return (table_hbm.at[idx_v.at[pl.ds(j * CHUNK, CHUNK)]],
                    rows_v.at[b], gsem[b])

        def _writeback_args(j, b):
            return (rows_v.at[b],
                    out_hbm.at[pl.ds(base + j * CHUNK, CHUNK)], wsem[b])

        def gather_start(j, b):
            pltpu.async_copy(*_gather_args(j, b))

        def gather_wait(j, b):
            pltpu.make_async_copy(*_gather_args(j, b)).wait()

        def writeback_start(j, b):
            pltpu.async_copy(*_writeback_args(j, b))

        def writeback_wait(j, b):
            pltpu.make_async_copy(*_writeback_args(j, b)).wait()

        for b in range(NBUF):
            gather_start(b, b)

        def superstep(g, carry):
            for b in range(NBUF):
                j = g * NBUF + b
                gather_wait(j, b)
                writeback_start(j, b)
            for b in range(NBUF):
                j = g * NBUF + b
                writeback_wait(j, b)
                gather_start(j + NBUF, b)
            return carry

        lax.fori_loop(0, n_super - 1, superstep, 0)

        g = n_super - 1
        for b in range(NBUF):
            j = g * NBUF + b
            gather_wait(j, b)
            writeback_start(j, b)
        for b in range(NBUF):
            writeback_wait(g * NBUF + b, b)

    return emb


def kernel(tokens, table):
    shape = tokens.shape
    flat = tokens.reshape(-1).astype(jnp.int32)
    zero = lax.optimization_barrier(jnp.zeros((), jnp.float32))
    out = _build(flat.shape[0])(flat, table + zero)
    return out.reshape(*shape, EMBED) + zero


# speedup vs baseline: 1.5738x; 1.5738x over previous
import functools

import jax
import jax.numpy as jnp
from jax import lax
from jax.experimental import pallas as pl
from jax.experimental.pallas import tpu as pltpu
from jax.experimental.pallas import tpu_sc as plsc

EMBED = 64
WIDE = 128
NC = 2
NS = 16
NW = NC * NS

CHUNK = 128
NBUF = 5


@functools.lru_cache(maxsize=None)
def _build(n_tokens):
    b_per_w = n_tokens // NW
    n_chunks = b_per_w // CHUNK
    n_super = n_chunks // NBUF
    assert b_per_w * NW == n_tokens
    assert n_chunks * CHUNK == b_per_w
    assert n_super * NBUF == n_chunks
    mesh = plsc.VectorSubcoreMesh(core_axis_name="c", subcore_axis_name="s")

    @functools.partial(
        pl.kernel,
        mesh=mesh,
        out_type=jax.ShapeDtypeStruct((n_tokens, WIDE), jnp.float32),
        scratch_types=[
            pltpu.VMEM((b_per_w,), jnp.int32),
            pltpu.VMEM((NBUF, CHUNK, WIDE), jnp.float32),
        ] + [pltpu.SemaphoreType.DMA] * (2 * NBUF),
        compiler_params=pltpu.CompilerParams(use_tc_tiling_on_sc=True),
    )
    def emb(tok_hbm, table_hbm, out_hbm, idx_v, rows_v, *sems):
        gsem = sems[:NBUF]
        wsem = sems[NBUF:]
        wid = lax.axis_index("s") * NC + lax.axis_index("c")
        base = wid * b_per_w
        pltpu.sync_copy(tok_hbm.at[pl.ds(base, b_per_w)], idx_v)

        def _gather_args(j, b):
            return (table_hbm.at[idx_v.at[pl.ds(j * CHUNK, CHUNK)]],
                    rows_v.at[b], gsem[b])

        def _writeback_args(j, b):
            return (rows_v.at[b],
                    out_hbm.at[pl.ds(base + j * CHUNK, CHUNK)], wsem[b])

        def gather_start(j, b):
            pltpu.async_copy(*_gather_args(j, b))

        def gather_wait(j, b):
            pltpu.make_async_copy(*_gather_args(j, b)).wait()

        def writeback_start(j, b):
            pltpu.async_copy(*_writeback_args(j, b))

        def writeback_wait(j, b):
            pltpu.make_async_copy(*_writeback_args(j, b)).wait()

        for b in range(NBUF):
            gather_start(b, b)

        def superstep(g, carry):
            for b in range(NBUF):
                j = g * NBUF + b
                gather_wait(j, b)
                writeback_start(j, b)
            for b in range(NBUF):
                j = g * NBUF + b
                writeback_wait(j, b)
                gather_start(j + NBUF, b)
            return carry

        lax.fori_loop(0, n_super - 1, superstep, 0)

        g = n_super - 1
        for b in range(NBUF):
            j = g * NBUF + b
            gather_wait(j, b)
            writeback_start(j, b)
        for b in range(NBUF):
            writeback_wait(g * NBUF + b, b)

    return emb


def kernel(tokens, table):
    shape = tokens.shape
    flat = tokens.reshape(-1).astype(jnp.int32)
    table_wide = jnp.pad(table, ((0, 0), (0, WIDE - EMBED)))
    out = _build(flat.shape[0])(flat, table_wide)
    return out[:, :EMBED].reshape(*shape, EMBED)


# speedup vs baseline: 1.7116x; 1.0876x over previous
import functools

import jax
import jax.numpy as jnp
from jax import lax
from jax.experimental import pallas as pl
from jax.experimental.pallas import tpu as pltpu
from jax.experimental.pallas import tpu_sc as plsc

EMBED = 64
SENT = 50
UNIT = 2 * SENT
NC = 2
NS = 16
NW = NC * NS

NBUF = 5


@functools.lru_cache(maxsize=None)
def _build(n_tokens):
    n_sent = n_tokens // SENT
    n_units = n_sent // 2
    u_per_w = n_units // NW
    n_super = u_per_w // NBUF
    assert n_sent * SENT == n_tokens and n_units * 2 == n_sent
    assert u_per_w * NW == n_units and n_super * NBUF == u_per_w
    mesh = plsc.VectorSubcoreMesh(core_axis_name="c", subcore_axis_name="s")

    @functools.partial(
        pl.kernel,
        mesh=mesh,
        out_type=jax.ShapeDtypeStruct((n_sent, SENT, EMBED), jnp.float32),
        scratch_types=[
            pltpu.VMEM((u_per_w, UNIT), jnp.int32),
            pltpu.VMEM((NBUF, UNIT, EMBED), jnp.float32),
        ] + [pltpu.SemaphoreType.DMA] * (2 * NBUF),
        compiler_params=pltpu.CompilerParams(use_tc_tiling_on_sc=False),
    )
    def emb(tok_hbm, table_hbm, out_hbm, idx_v, rows_v, *sems):
        gsem = sems[:NBUF]
        wsem = sems[NBUF:]
        wid = lax.axis_index("s") * NC + lax.axis_index("c")
        base = wid * u_per_w
        pltpu.sync_copy(tok_hbm.at[pl.ds(base, u_per_w)], idx_v)

        def _gather_args(u, b):
            return (table_hbm.at[idx_v.at[u]], rows_v.at[b], gsem[b])

        def _wb_args(u, b, half):
            return (rows_v.at[b].at[pl.ds(half * SENT, SENT)],
                    out_hbm.at[2 * (base + u) + half], wsem[b])

        def gather_start(u, b):
            pltpu.async_copy(*_gather_args(u, b))

        def gather_wait(u, b):
            pltpu.make_async_copy(*_gather_args(u, b)).wait()

        def wb_start(u, b):
            pltpu.async_copy(*_wb_args(u, b, 0))
            pltpu.async_copy(*_wb_args(u, b, 1))

        def wb_wait(u, b):
            pltpu.make_async_copy(*_wb_args(u, b, 0)).wait()
            pltpu.make_async_copy(*_wb_args(u, b, 1)).wait()

        for b in range(NBUF):
            gather_start(b, b)

        def superstep(g, carry):
            for b in range(NBUF):
                u = g * NBUF + b
                gather_wait(u, b)
                wb_start(u, b)
            for b in range(NBUF):
                u = g * NBUF + b
                wb_wait(u, b)
                gather_start(u + NBUF, b)
            return carry

        lax.fori_loop(0, n_super - 1, superstep, 0)

        g = n_super - 1
        for b in range(NBUF):
            u = g * NBUF + b
            gather_wait(u, b)
            wb_start(u, b)
        for b in range(NBUF):
            wb_wait(g * NBUF + b, b)

    return emb


def kernel(tokens, table):
    shape = tokens.shape
    n_tokens = tokens.size
    units = tokens.reshape(n_tokens // UNIT, UNIT).astype(jnp.int32)
    out = _build(n_tokens)(units, table)
    return out.reshape(*shape, EMBED)
